# baseline (speedup 1.0000x reference)
import functools

import jax
import jax.numpy as jnp
from jax import lax
from jax.experimental import pallas as pl
from jax.experimental.pallas import tpu as pltpu
from jax.experimental.pallas import tpu_sc as plsc

F32 = jnp.float32


def _tc_prep1(x, W1, S, Dm):
    n, d = x.shape
    f1 = W1.shape[1]
    blk = 256
    grid = n // blk

    def body(x_ref, w_ref, s_ref, dm_ref, out_ref):
        h1 = jnp.dot(x_ref[...], w_ref[...], preferred_element_type=F32)
        as1 = jnp.dot(h1, s_ref[...], preferred_element_type=F32)
        ad1 = jnp.dot(h1, dm_ref[...], preferred_element_type=F32)
        pad = jnp.zeros((blk, 56), F32)
        out_ref[...] = jnp.concatenate([h1, as1, ad1, pad], axis=1)

    return pl.pallas_call(
        body,
        grid=(grid,),
        in_specs=[
            pl.BlockSpec((blk, d), lambda i: (i, 0)),
            pl.BlockSpec((d, f1), lambda i: (0, 0)),
            pl.BlockSpec((f1, 4), lambda i: (0, 0)),
            pl.BlockSpec((f1, 4), lambda i: (0, 0)),
        ],
        out_specs=pl.BlockSpec((blk, 128), lambda i: (i, 0)),
        out_shape=jax.ShapeDtypeStruct((n, 128), F32),
    )(x, W1, S, Dm)


def _sc_pass1(P1, src, dst, zrows, n, e_total):
    info = plsc.get_sparse_core_info()
    nc, ns = info.num_cores, info.num_subcores
    ept = e_total // (nc * ns)
    K = 32
    nb = ept // K
    hrows = n // 2
    erows = n // 16
    hrpt = hrows // ns
    erpt = erows // ns

    mesh = plsc.VectorSubcoreMesh(core_axis_name="c", subcore_axis_name="s")

    @functools.partial(
        pl.kernel,
        out_type=[
            jax.ShapeDtypeStruct((nc, hrows, 128), F32),
            jax.ShapeDtypeStruct((nc, erows, 128), F32),
        ],
        mesh=mesh,
        scratch_types=[
            pltpu.VMEM((2, K), jnp.int32),
            pltpu.VMEM((2, K), jnp.int32),
            pltpu.VMEM((2, K + 16), jnp.int32),
            pltpu.VMEM((2, K), jnp.int32),
            pltpu.VMEM((2, K), jnp.int32),
            pltpu.VMEM((2, K, 128), F32),
            pltpu.VMEM((2, K, 128), F32),
            pltpu.VMEM((2, K, 128), F32),
            pltpu.VMEM((2, K, 128), F32),
            pltpu.VMEM_SHARED((hrows, 128), F32),
            pltpu.VMEM_SHARED((erows, 128), F32),
            pltpu.SemaphoreType.DMA,
            pltpu.SemaphoreType.DMA,
            pltpu.SemaphoreType.DMA,
            pltpu.SemaphoreType.DMA,
        ],
    )
    def k(p1_h, src_h, dst_h, z_h, outh_h, oute_h,
          idx_s, idx_d, idx_dp, idx_h, idx_e, gbuf_s, gbuf_d, obuf_h, obuf_e,
          acc_h, acc_e, sg0, sg1, ss0, ss1):
        c = lax.axis_index("c")
        s = lax.axis_index("s")
        pltpu.sync_copy(z_h, acc_h.at[pl.ds(s * hrpt, hrpt)])
        pltpu.sync_copy(z_h.at[pl.ds(0, erpt)], acc_e.at[pl.ds(s * erpt, erpt)])
        pltpu.sync_copy(z_h.at[pl.ds(0, K)], obuf_e.at[0])
        pltpu.sync_copy(z_h.at[pl.ds(0, K)], obuf_e.at[1])
        plsc.subcore_barrier()

        lanes = lax.iota(jnp.int32, 16)
        zv = jnp.zeros((16,), F32)
        base = (c * ns + s) * ept
        sg = (sg0, sg1)
        ss = (ss0, ss1)

        def issue_batch(slot, off):
            pltpu.sync_copy(src_h.at[pl.ds(off, K)], idx_s.at[slot])
            pltpu.sync_copy(dst_h.at[pl.ds(off, K)], idx_d.at[slot])
            pltpu.sync_copy(dst_h.at[pl.ds(off, K)],
                            idx_dp.at[slot, pl.ds(0, K)])
            pltpu.async_copy(p1_h.at[idx_s.at[slot]], gbuf_s.at[slot],
                             sg[slot])
            pltpu.async_copy(p1_h.at[idx_d.at[slot]], gbuf_d.at[slot],
                             sg[slot])

        def wait_gathers(slot):
            pltpu.make_async_copy(p1_h.at[idx_s.at[slot]], gbuf_s.at[slot],
                                  sg[slot]).wait()
            pltpu.make_async_copy(p1_h.at[idx_d.at[slot]], gbuf_d.at[slot],
                                  sg[slot]).wait()

        def issue_scatters(slot):
            pltpu.async_copy(obuf_h.at[slot], acc_h.at[idx_h.at[slot]],
                             ss[slot], add=True)
            pltpu.async_copy(obuf_e.at[slot], acc_e.at[idx_e.at[slot]],
                             ss[slot], add=True)

        def wait_scatters(slot):
            pltpu.make_async_copy(obuf_h.at[slot], acc_h.at[idx_h.at[slot]],
                                  ss[slot]).wait()
            pltpu.make_async_copy(obuf_e.at[slot], acc_e.at[idx_e.at[slot]],
                                  ss[slot]).wait()

        def compute(slot):
            for j in range(K // 16):
                dv = idx_d[slot, pl.ds(j * 16, 16)]
                idx_h[slot, pl.ds(j * 16, 16)] = lax.shift_right_logical(dv, 1)
                idx_e[slot, pl.ds(j * 16, 16)] = lax.shift_right_logical(dv, 4)

            @plsc.parallel_loop(0, K, 1, unroll=8)
            def edge(i):
                d = idx_dp[slot, pl.ds(i, 16)][0]
                av = (gbuf_s[slot, i, pl.ds(64, 16)]
                      + gbuf_d[slot, i, pl.ds(68, 16)])
                av = jnp.maximum(av, 0.2 * av)
                e = jnp.exp(av)
                e = jnp.where(lanes < 4, e, 0.0)
                half = (d & 1) * 64
                ohalf = 64 - half
                obuf_h[slot, i, pl.ds(half + 0, 16)] = (
                    gbuf_s[slot, i, pl.ds(0, 16)] * e[0])
                obuf_h[slot, i, pl.ds(half + 16, 16)] = (
                    gbuf_s[slot, i, pl.ds(16, 16)] * e[1])
                obuf_h[slot, i, pl.ds(half + 32, 16)] = (
                    gbuf_s[slot, i, pl.ds(32, 16)] * e[2])
                obuf_h[slot, i, pl.ds(half + 48, 16)] = (
                    gbuf_s[slot, i, pl.ds(48, 16)] * e[3])
                obuf_h[slot, i, pl.ds(ohalf + 0, 16)] = zv
                obuf_h[slot, i, pl.ds(ohalf + 16, 16)] = zv
                obuf_h[slot, i, pl.ds(ohalf + 32, 16)] = zv
                obuf_h[slot, i, pl.ds(ohalf + 48, 16)] = zv
                obuf_e[slot, i, pl.ds(0, 16)] = zv
                obuf_e[slot, i, pl.ds(16, 16)] = zv
                obuf_e[slot, i, pl.ds(32, 16)] = zv
                obuf_e[slot, i, pl.ds(48, 16)] = zv
                col = (d & 15) * 4
                obuf_e[slot, i, pl.ds(col, 16)] = e

        nb2 = nb // 2
        issue_batch(0, base)

        def run_pair(bb, carry):
            b0 = 2 * bb
            issue_batch(1, base + (b0 + 1) * K)
            wait_gathers(0)
            pl.when(bb > 0)(lambda: wait_scatters(0))
            compute(0)
            issue_scatters(0)
            pl.when(bb + 1 < nb2)(lambda: issue_batch(0, base + (b0 + 2) * K))
            wait_gathers(1)
            pl.when(bb > 0)(lambda: wait_scatters(1))
            compute(1)
            issue_scatters(1)
            return carry

        lax.fori_loop(0, nb2, run_pair, 0)
        wait_scatters(0)
        wait_scatters(1)
        plsc.subcore_barrier()
        pltpu.sync_copy(acc_h.at[pl.ds(s * hrpt, hrpt)],
                        outh_h.at[c, pl.ds(s * hrpt, hrpt)])
        pltpu.sync_copy(acc_e.at[pl.ds(s * erpt, erpt)],
                        oute_h.at[c, pl.ds(s * erpt, erpt)])

    return k(P1, src, dst, zrows)


def _tc_mid(T0, T1, D0, D1, P1, R, b1, W2):
    n = P1.shape[0]
    f1 = W2.shape[0]
    blk = 256
    grid = n // blk

    def body(t0_ref, t1_ref, d0_ref, d1_ref, p1_ref, r_ref, b1_ref, w2_ref,
             h2_ref):
        h1 = p1_ref[:, 0:64]
        as1 = p1_ref[:, 64:68]
        ad1 = p1_ref[:, 68:72]
        al = as1 + ad1
        al = jnp.maximum(al, 0.2 * al)
        es = jnp.exp(al)
        es64 = jnp.dot(es, r_ref[...], preferred_element_type=F32)
        num = t0_ref[...] + t1_ref[...] + h1 * es64
        den = d0_ref[...] + d1_ref[...] + es
        den64 = jnp.dot(den, r_ref[...], preferred_element_type=F32)
        g1 = jnp.maximum(num / (den64 + 1e-16) + b1_ref[...], 0.0)
        h2_ref[...] = jnp.dot(g1, w2_ref[...], preferred_element_type=F32)

    return pl.pallas_call(
        body,
        grid=(grid,),
        in_specs=[
            pl.BlockSpec((blk, 64), lambda i: (i, 0)),
            pl.BlockSpec((blk, 64), lambda i: (i, 0)),
            pl.BlockSpec((blk, 4), lambda i: (i, 0)),
            pl.BlockSpec((blk, 4), lambda i: (i, 0)),
            pl.BlockSpec((blk, 128), lambda i: (i, 0)),
            pl.BlockSpec((4, 64), lambda i: (0, 0)),
            pl.BlockSpec((1, 64), lambda i: (0, 0)),
            pl.BlockSpec((f1, 1), lambda i: (0, 0)),
        ],
        out_specs=pl.BlockSpec((blk, 1), lambda i: (i, 0)),
        out_shape=jax.ShapeDtypeStruct((n, 1), F32),
    )(T0, T1, D0, D1, P1, R, b1, W2)


def _sc_pass2(h2f, cvec, src, dst, zrows, n, e_total):
    info = plsc.get_sparse_core_info()
    nc, ns = info.num_cores, info.num_subcores
    ept = e_total // (nc * ns)
    K = 128
    nb = ept // K
    arows = n // 8
    arpt = arows // ns

    mesh = plsc.VectorSubcoreMesh(core_axis_name="c", subcore_axis_name="s")

    @functools.partial(
        pl.kernel,
        out_type=jax.ShapeDtypeStruct((nc, arows, 128), F32),
        mesh=mesh,
        compiler_params=pltpu.CompilerParams(needs_layout_passes=False,
                                             use_tc_tiling_on_sc=False),
        scratch_types=[
            pltpu.VMEM((K,), jnp.int32),
            pltpu.VMEM((K,), jnp.int32),
            pltpu.VMEM((K,), jnp.int32),
            pltpu.VMEM((K, 128), F32),
            pltpu.VMEM((n // 16, 16), F32),
            pltpu.VMEM((16,), F32),
            pltpu.VMEM_SHARED((arows, 128), F32),
            pltpu.SemaphoreType.DMA,
        ],
    )
    def k(h2_h, cv_h, src_h, dst_h, z_h, out_h,
          idx_s, idx_d, idx_r, obuf, h2t, cbuf, acc, sem):
        c = lax.axis_index("c")
        s = lax.axis_index("s")
        pltpu.sync_copy(z_h.at[pl.ds(0, arpt)], acc.at[pl.ds(s * arpt, arpt)])
        pltpu.sync_copy(z_h, obuf)
        pltpu.sync_copy(h2_h, h2t)
        pltpu.sync_copy(cv_h, cbuf)
        plsc.subcore_barrier()

        lanes = lax.iota(jnp.int32, 16)
        zv = jnp.zeros((16,), F32)
        cb = cbuf[...]
        c1 = cb[0]
        c2 = cb[1]
        base = (c * ns + s) * ept

        def run_batch(b, carry):
            off = base + b * K
            pltpu.sync_copy(src_h.at[pl.ds(off, K)], idx_s)
            pltpu.sync_copy(dst_h.at[pl.ds(off, K)], idx_d)
            for j in range(K // 16):
                rows = lanes + (j * 16)
                sv = idx_s[pl.ds(j * 16, 16)]
                dv = idx_d[pl.ds(j * 16, 16)]
                idx_r[pl.ds(j * 16, 16)] = lax.shift_right_logical(dv, 3)
                h2s = plsc.load_gather(
                    h2t, [lax.shift_right_logical(sv, 4), sv & 15])
                h2d = plsc.load_gather(
                    h2t, [lax.shift_right_logical(dv, 4), dv & 15])
                av = c1 * h2s + c2 * h2d
                av = jnp.maximum(av, 0.2 * av)
                e = jnp.exp(av)
                colv = (dv & 7) * 16
                plsc.store_scatter(obuf, [rows, colv], e * h2s)
                plsc.store_scatter(obuf, [rows, colv + 1], e)
            pltpu.sync_copy(obuf, acc.at[idx_r], add=True)
            for j in range(K // 16):
                rows = lanes + (j * 16)
                dv = idx_d[pl.ds(j * 16, 16)]
                colv = (dv & 7) * 16
                plsc.store_scatter(obuf, [rows, colv], zv)
                plsc.store_scatter(obuf, [rows, colv + 1], zv)
            return carry

        lax.fori_loop(0, nb, run_batch, 0)
        plsc.subcore_barrier()
        pltpu.sync_copy(acc.at[pl.ds(s * arpt, arpt)],
                        out_h.at[c, pl.ds(s * arpt, arpt)])

    return k(h2f, cvec, src, dst, zrows)


def _tc_final(x, WT, b_emb, h2c, asc2, adc2, U0, U1, maskf, b2, bsz, lsz):
    n, d = x.shape

    def body(x_ref, wt_ref, be_ref, h2_ref, asc_ref, adc_ref, u0_ref, u1_ref,
             m_ref, b2_ref, out1_ref, xo_ref):
        h2 = h2_ref[...]
        al = h2 * (asc_ref[0, 0] + adc_ref[0, 0])
        al = jnp.maximum(al, 0.2 * al)
        es = jnp.exp(al)
        num = u0_ref[:, 0:1] + u1_ref[:, 0:1] + es * h2
        den = u0_ref[:, 1:2] + u1_ref[:, 1:2] + es
        z = num / (den + 1e-16) + b2_ref[0, 0]
        att = 1.0 / (1.0 + jnp.exp(-z))
        emb = jnp.dot(x_ref[...], wt_ref[...], preferred_element_type=F32)
        emb = jnp.maximum(emb + be_ref[...], 0.0)
        xv = att * emb
        xo_ref[...] = xv
        m = m_ref[...]
        pmax = jnp.max(xv + (m - 1.0) * 1e9, axis=0)
        pmean = jnp.sum(xv * m, axis=0) / jnp.sum(m)
        g = pl.program_id(0)
        out1_ref[pl.ds(g, 1), :] = (pmax + pmean)[None, :]

    return pl.pallas_call(
        body,
        grid=(bsz,),
        in_specs=[
            pl.BlockSpec((lsz, d), lambda i: (i, 0)),
            pl.BlockSpec((d, d), lambda i: (0, 0)),
            pl.BlockSpec((1, d), lambda i: (0, 0)),
            pl.BlockSpec((lsz, 1), lambda i: (i, 0)),
            pl.BlockSpec((1, 1), lambda i: (0, 0)),
            pl.BlockSpec((1, 1), lambda i: (0, 0)),
            pl.BlockSpec((lsz, 16), lambda i: (i, 0)),
            pl.BlockSpec((lsz, 16), lambda i: (i, 0)),
            pl.BlockSpec((lsz, 1), lambda i: (i, 0)),
            pl.BlockSpec((1, 1), lambda i: (0, 0)),
        ],
        out_specs=[
            pl.BlockSpec((bsz, d), lambda i: (0, 0)),
            pl.BlockSpec((lsz, d), lambda i: (i, 0)),
        ],
        out_shape=[
            jax.ShapeDtypeStruct((bsz, d), F32),
            jax.ShapeDtypeStruct((n, d), F32),
        ],
    )(x, WT, b_emb, h2c, asc2, adc2, U0, U1, maskf, b2)


def kernel(x, mask, edge_index, length, W1, a_src1, a_dst1, b1,
           W2, a_src2, a_dst2, b2, W_emb, b_emb):
    n, d = x.shape
    e_total = edge_index.shape[1]
    bsz = length.shape[0]
    lsz = n // bsz
    h_heads, c_ch = a_src1.shape[1], a_src1.shape[2]
    f1 = W1.shape[1]

    src = edge_index[0]
    dst = edge_index[1]

    eye = jnp.eye(h_heads, dtype=F32)
    S = (a_src1[0][:, :, None] * eye[:, None, :]).reshape(f1, h_heads)
    Dm = (a_dst1[0][:, :, None] * eye[:, None, :]).reshape(f1, h_heads)
    R = jnp.repeat(eye, c_ch, axis=1)

    P1 = _tc_prep1(x, W1, S, Dm)

    z1 = jnp.zeros((n // 2 // 16, 128), F32)
    Th, Te = _sc_pass1(P1, src, dst, z1, n, e_total)

    T0 = Th[0].reshape(n, 64)
    T1 = Th[1].reshape(n, 64)
    D0 = Te[0][:, 0:64].reshape(n, 4)
    D1 = Te[1][:, 0:64].reshape(n, 4)

    h2c = _tc_mid(T0, T1, D0, D1, P1, R, b1.reshape(1, f1), W2)

    cvec = jnp.concatenate(
        [a_src2.reshape(1), a_dst2.reshape(1), jnp.zeros((14,), F32)])
    z2 = jnp.zeros((128, 128), F32)
    U = _sc_pass2(h2c.reshape(n // 16, 16), cvec, src, dst, z2, n, e_total)
    U0 = U[0].reshape(n, 16)
    U1 = U[1].reshape(n, 16)

    asc2 = a_src2.reshape(1, 1)
    adc2 = a_dst2.reshape(1, 1)
    out1, x_ = _tc_final(x, W_emb.T, b_emb.reshape(1, d), h2c, asc2, adc2,
                         U0, U1, mask.reshape(n, 1), b2.reshape(1, 1),
                         bsz, lsz)
    return (out1, x_)


# speedup vs baseline: 1.0023x; 1.0023x over previous
import functools

import jax
import jax.numpy as jnp
from jax import lax
from jax.experimental import pallas as pl
from jax.experimental.pallas import tpu as pltpu
from jax.experimental.pallas import tpu_sc as plsc

F32 = jnp.float32


def _tc_prep1(x, W1, S, Dm):
    n, d = x.shape
    f1 = W1.shape[1]
    blk = 256
    grid = n // blk

    def body(x_ref, w_ref, s_ref, dm_ref, out_ref):
        h1 = jnp.dot(x_ref[...], w_ref[...], preferred_element_type=F32)
        as1 = jnp.dot(h1, s_ref[...], preferred_element_type=F32)
        ad1 = jnp.dot(h1, dm_ref[...], preferred_element_type=F32)
        pad = jnp.zeros((blk, 56), F32)
        out_ref[...] = jnp.concatenate([h1, as1, ad1, pad], axis=1)

    return pl.pallas_call(
        body,
        grid=(grid,),
        in_specs=[
            pl.BlockSpec((blk, d), lambda i: (i, 0)),
            pl.BlockSpec((d, f1), lambda i: (0, 0)),
            pl.BlockSpec((f1, 4), lambda i: (0, 0)),
            pl.BlockSpec((f1, 4), lambda i: (0, 0)),
        ],
        out_specs=pl.BlockSpec((blk, 128), lambda i: (i, 0)),
        out_shape=jax.ShapeDtypeStruct((n, 128), F32),
    )(x, W1, S, Dm)


def _sc_pass1(P1, src, dst, zrows, n, e_total):
    info = plsc.get_sparse_core_info()
    nc, ns = info.num_cores, info.num_subcores
    ept = e_total // (nc * ns)
    K = 32
    nb = ept // K
    hrows = n // 2
    erows = n // 16
    hrpt = hrows // ns
    erpt = erows // ns

    mesh = plsc.VectorSubcoreMesh(core_axis_name="c", subcore_axis_name="s")

    @functools.partial(
        pl.kernel,
        out_type=[
            jax.ShapeDtypeStruct((nc, hrows, 128), F32),
            jax.ShapeDtypeStruct((nc, erows, 128), F32),
        ],
        mesh=mesh,
        scratch_types=[
            pltpu.VMEM((2, K), jnp.int32),
            pltpu.VMEM((2, K), jnp.int32),
            pltpu.VMEM((2, K + 16), jnp.int32),
            pltpu.VMEM((2, K), jnp.int32),
            pltpu.VMEM((2, K), jnp.int32),
            pltpu.VMEM((2, K, 128), F32),
            pltpu.VMEM((2, K, 128), F32),
            pltpu.VMEM((2, K, 128), F32),
            pltpu.VMEM((2, K, 128), F32),
            pltpu.VMEM_SHARED((hrows, 128), F32),
            pltpu.VMEM_SHARED((erows, 128), F32),
            pltpu.SemaphoreType.DMA,
            pltpu.SemaphoreType.DMA,
            pltpu.SemaphoreType.DMA,
            pltpu.SemaphoreType.DMA,
        ],
    )
    def k(p1_h, src_h, dst_h, z_h, outh_h, oute_h,
          idx_s, idx_d, idx_dp, idx_h, idx_e, gbuf_s, gbuf_d, obuf_h, obuf_e,
          acc_h, acc_e, sg0, sg1, ss0, ss1):
        c = lax.axis_index("c")
        s = lax.axis_index("s")
        pltpu.sync_copy(z_h, acc_h.at[pl.ds(s * hrpt, hrpt)])
        pltpu.sync_copy(z_h.at[pl.ds(0, erpt)], acc_e.at[pl.ds(s * erpt, erpt)])
        pltpu.sync_copy(z_h.at[pl.ds(0, K)], obuf_e.at[0])
        pltpu.sync_copy(z_h.at[pl.ds(0, K)], obuf_e.at[1])
        plsc.subcore_barrier()

        lanes = lax.iota(jnp.int32, 16)
        zv = jnp.zeros((16,), F32)
        base = (c * ns + s) * ept
        sg = (sg0, sg1)
        ss = (ss0, ss1)

        def issue_batch(slot, off):
            pltpu.sync_copy(src_h.at[pl.ds(off, K)], idx_s.at[slot])
            pltpu.sync_copy(dst_h.at[pl.ds(off, K)], idx_d.at[slot])
            pltpu.sync_copy(dst_h.at[pl.ds(off, K)],
                            idx_dp.at[slot, pl.ds(0, K)])
            pltpu.async_copy(p1_h.at[idx_s.at[slot]], gbuf_s.at[slot],
                             sg[slot])
            pltpu.async_copy(p1_h.at[idx_d.at[slot]], gbuf_d.at[slot],
                             sg[slot])

        def wait_gathers(slot):
            pltpu.make_async_copy(p1_h.at[idx_s.at[slot]], gbuf_s.at[slot],
                                  sg[slot]).wait()
            pltpu.make_async_copy(p1_h.at[idx_d.at[slot]], gbuf_d.at[slot],
                                  sg[slot]).wait()

        def issue_scatters(slot):
            pltpu.async_copy(obuf_h.at[slot], acc_h.at[idx_h.at[slot]],
                             ss[slot], add=True)
            pltpu.async_copy(obuf_e.at[slot], acc_e.at[idx_e.at[slot]],
                             ss[slot], add=True)

        def wait_scatters(slot):
            pltpu.make_async_copy(obuf_h.at[slot], acc_h.at[idx_h.at[slot]],
                                  ss[slot]).wait()
            pltpu.make_async_copy(obuf_e.at[slot], acc_e.at[idx_e.at[slot]],
                                  ss[slot]).wait()

        def compute(slot):
            for j in range(K // 16):
                dv = idx_d[slot, pl.ds(j * 16, 16)]
                idx_h[slot, pl.ds(j * 16, 16)] = lax.shift_right_logical(dv, 1)
                idx_e[slot, pl.ds(j * 16, 16)] = lax.shift_right_logical(dv, 4)

            @plsc.parallel_loop(0, K, 1, unroll=4)
            def edge(i):
                d = idx_dp[slot, pl.ds(i, 16)][0]
                av = (gbuf_s[slot, i, pl.ds(64, 16)]
                      + gbuf_d[slot, i, pl.ds(68, 16)])
                av = jnp.maximum(av, 0.2 * av)
                e = jnp.exp(av)
                e = jnp.where(lanes < 4, e, 0.0)
                half = (d & 1) * 64
                ohalf = 64 - half
                obuf_h[slot, i, pl.ds(half + 0, 16)] = (
                    gbuf_s[slot, i, pl.ds(0, 16)] * e[0])
                obuf_h[slot, i, pl.ds(half + 16, 16)] = (
                    gbuf_s[slot, i, pl.ds(16, 16)] * e[1])
                obuf_h[slot, i, pl.ds(half + 32, 16)] = (
                    gbuf_s[slot, i, pl.ds(32, 16)] * e[2])
                obuf_h[slot, i, pl.ds(half + 48, 16)] = (
                    gbuf_s[slot, i, pl.ds(48, 16)] * e[3])
                obuf_h[slot, i, pl.ds(ohalf + 0, 16)] = zv
                obuf_h[slot, i, pl.ds(ohalf + 16, 16)] = zv
                obuf_h[slot, i, pl.ds(ohalf + 32, 16)] = zv
                obuf_h[slot, i, pl.ds(ohalf + 48, 16)] = zv
                obuf_e[slot, i, pl.ds(0, 16)] = zv
                obuf_e[slot, i, pl.ds(16, 16)] = zv
                obuf_e[slot, i, pl.ds(32, 16)] = zv
                obuf_e[slot, i, pl.ds(48, 16)] = zv
                col = (d & 15) * 4
                obuf_e[slot, i, pl.ds(col, 16)] = e

        nb2 = nb // 2
        issue_batch(0, base)

        def run_pair(bb, carry):
            b0 = 2 * bb
            issue_batch(1, base + (b0 + 1) * K)
            wait_gathers(0)
            pl.when(bb > 0)(lambda: wait_scatters(0))
            compute(0)
            issue_scatters(0)
            pl.when(bb + 1 < nb2)(lambda: issue_batch(0, base + (b0 + 2) * K))
            wait_gathers(1)
            pl.when(bb > 0)(lambda: wait_scatters(1))
            compute(1)
            issue_scatters(1)
            return carry

        lax.fori_loop(0, nb2, run_pair, 0)
        wait_scatters(0)
        wait_scatters(1)
        plsc.subcore_barrier()
        pltpu.sync_copy(acc_h.at[pl.ds(s * hrpt, hrpt)],
                        outh_h.at[c, pl.ds(s * hrpt, hrpt)])
        pltpu.sync_copy(acc_e.at[pl.ds(s * erpt, erpt)],
                        oute_h.at[c, pl.ds(s * erpt, erpt)])

    return k(P1, src, dst, zrows)


def _tc_mid(T0, T1, D0, D1, P1, R, b1, W2):
    n = P1.shape[0]
    f1 = W2.shape[0]
    blk = 256
    grid = n // blk

    def body(t0_ref, t1_ref, d0_ref, d1_ref, p1_ref, r_ref, b1_ref, w2_ref,
             h2_ref):
        h1 = p1_ref[:, 0:64]
        as1 = p1_ref[:, 64:68]
        ad1 = p1_ref[:, 68:72]
        al = as1 + ad1
        al = jnp.maximum(al, 0.2 * al)
        es = jnp.exp(al)
        es64 = jnp.dot(es, r_ref[...], preferred_element_type=F32)
        num = t0_ref[...] + t1_ref[...] + h1 * es64
        den = d0_ref[...] + d1_ref[...] + es
        den64 = jnp.dot(den, r_ref[...], preferred_element_type=F32)
        g1 = jnp.maximum(num / (den64 + 1e-16) + b1_ref[...], 0.0)
        h2_ref[...] = jnp.dot(g1, w2_ref[...], preferred_element_type=F32)

    return pl.pallas_call(
        body,
        grid=(grid,),
        in_specs=[
            pl.BlockSpec((blk, 64), lambda i: (i, 0)),
            pl.BlockSpec((blk, 64), lambda i: (i, 0)),
            pl.BlockSpec((blk, 4), lambda i: (i, 0)),
            pl.BlockSpec((blk, 4), lambda i: (i, 0)),
            pl.BlockSpec((blk, 128), lambda i: (i, 0)),
            pl.BlockSpec((4, 64), lambda i: (0, 0)),
            pl.BlockSpec((1, 64), lambda i: (0, 0)),
            pl.BlockSpec((f1, 1), lambda i: (0, 0)),
        ],
        out_specs=pl.BlockSpec((blk, 1), lambda i: (i, 0)),
        out_shape=jax.ShapeDtypeStruct((n, 1), F32),
    )(T0, T1, D0, D1, P1, R, b1, W2)


def _sc_pass2(h2f, cvec, src, dst, zrows, n, e_total):
    info = plsc.get_sparse_core_info()
    nc, ns = info.num_cores, info.num_subcores
    ept = e_total // (nc * ns)
    K = 128
    nb = ept // K
    arows = n // 8
    arpt = arows // ns

    mesh = plsc.VectorSubcoreMesh(core_axis_name="c", subcore_axis_name="s")

    @functools.partial(
        pl.kernel,
        out_type=jax.ShapeDtypeStruct((nc, arows, 128), F32),
        mesh=mesh,
        compiler_params=pltpu.CompilerParams(needs_layout_passes=False,
                                             use_tc_tiling_on_sc=False),
        scratch_types=[
            pltpu.VMEM((K,), jnp.int32),
            pltpu.VMEM((K,), jnp.int32),
            pltpu.VMEM((K,), jnp.int32),
            pltpu.VMEM((K, 128), F32),
            pltpu.VMEM((n // 16, 16), F32),
            pltpu.VMEM((16,), F32),
            pltpu.VMEM_SHARED((arows, 128), F32),
            pltpu.SemaphoreType.DMA,
        ],
    )
    def k(h2_h, cv_h, src_h, dst_h, z_h, out_h,
          idx_s, idx_d, idx_r, obuf, h2t, cbuf, acc, sem):
        c = lax.axis_index("c")
        s = lax.axis_index("s")
        pltpu.sync_copy(z_h.at[pl.ds(0, arpt)], acc.at[pl.ds(s * arpt, arpt)])
        pltpu.sync_copy(z_h, obuf)
        pltpu.sync_copy(h2_h, h2t)
        pltpu.sync_copy(cv_h, cbuf)
        plsc.subcore_barrier()

        lanes = lax.iota(jnp.int32, 16)
        zv = jnp.zeros((16,), F32)
        cb = cbuf[...]
        c1 = cb[0]
        c2 = cb[1]
        base = (c * ns + s) * ept

        def run_batch(b, carry):
            off = base + b * K
            pltpu.sync_copy(src_h.at[pl.ds(off, K)], idx_s)
            pltpu.sync_copy(dst_h.at[pl.ds(off, K)], idx_d)
            for j in range(K // 16):
                rows = lanes + (j * 16)
                sv = idx_s[pl.ds(j * 16, 16)]
                dv = idx_d[pl.ds(j * 16, 16)]
                idx_r[pl.ds(j * 16, 16)] = lax.shift_right_logical(dv, 3)
                h2s = plsc.load_gather(
                    h2t, [lax.shift_right_logical(sv, 4), sv & 15])
                h2d = plsc.load_gather(
                    h2t, [lax.shift_right_logical(dv, 4), dv & 15])
                av = c1 * h2s + c2 * h2d
                av = jnp.maximum(av, 0.2 * av)
                e = jnp.exp(av)
                colv = (dv & 7) * 16
                plsc.store_scatter(obuf, [rows, colv], e * h2s)
                plsc.store_scatter(obuf, [rows, colv + 1], e)
            pltpu.sync_copy(obuf, acc.at[idx_r], add=True)
            for j in range(K // 16):
                rows = lanes + (j * 16)
                dv = idx_d[pl.ds(j * 16, 16)]
                colv = (dv & 7) * 16
                plsc.store_scatter(obuf, [rows, colv], zv)
                plsc.store_scatter(obuf, [rows, colv + 1], zv)
            return carry

        lax.fori_loop(0, nb, run_batch, 0)
        plsc.subcore_barrier()
        pltpu.sync_copy(acc.at[pl.ds(s * arpt, arpt)],
                        out_h.at[c, pl.ds(s * arpt, arpt)])

    return k(h2f, cvec, src, dst, zrows)


def _tc_final(x, WT, b_emb, h2c, asc2, adc2, U0, U1, maskf, b2, bsz, lsz):
    n, d = x.shape

    def body(x_ref, wt_ref, be_ref, h2_ref, asc_ref, adc_ref, u0_ref, u1_ref,
             m_ref, b2_ref, out1_ref, xo_ref):
        h2 = h2_ref[...]
        al = h2 * (asc_ref[0, 0] + adc_ref[0, 0])
        al = jnp.maximum(al, 0.2 * al)
        es = jnp.exp(al)
        num = u0_ref[:, 0:1] + u1_ref[:, 0:1] + es * h2
        den = u0_ref[:, 1:2] + u1_ref[:, 1:2] + es
        z = num / (den + 1e-16) + b2_ref[0, 0]
        att = 1.0 / (1.0 + jnp.exp(-z))
        emb = jnp.dot(x_ref[...], wt_ref[...], preferred_element_type=F32)
        emb = jnp.maximum(emb + be_ref[...], 0.0)
        xv = att * emb
        xo_ref[...] = xv
        m = m_ref[...]
        pmax = jnp.max(xv + (m - 1.0) * 1e9, axis=0)
        pmean = jnp.sum(xv * m, axis=0) / jnp.sum(m)
        g = pl.program_id(0)
        out1_ref[pl.ds(g, 1), :] = (pmax + pmean)[None, :]

    return pl.pallas_call(
        body,
        grid=(bsz,),
        in_specs=[
            pl.BlockSpec((lsz, d), lambda i: (i, 0)),
            pl.BlockSpec((d, d), lambda i: (0, 0)),
            pl.BlockSpec((1, d), lambda i: (0, 0)),
            pl.BlockSpec((lsz, 1), lambda i: (i, 0)),
            pl.BlockSpec((1, 1), lambda i: (0, 0)),
            pl.BlockSpec((1, 1), lambda i: (0, 0)),
            pl.BlockSpec((lsz, 16), lambda i: (i, 0)),
            pl.BlockSpec((lsz, 16), lambda i: (i, 0)),
            pl.BlockSpec((lsz, 1), lambda i: (i, 0)),
            pl.BlockSpec((1, 1), lambda i: (0, 0)),
        ],
        out_specs=[
            pl.BlockSpec((bsz, d), lambda i: (0, 0)),
            pl.BlockSpec((lsz, d), lambda i: (i, 0)),
        ],
        out_shape=[
            jax.ShapeDtypeStruct((bsz, d), F32),
            jax.ShapeDtypeStruct((n, d), F32),
        ],
    )(x, WT, b_emb, h2c, asc2, adc2, U0, U1, maskf, b2)


def kernel(x, mask, edge_index, length, W1, a_src1, a_dst1, b1,
           W2, a_src2, a_dst2, b2, W_emb, b_emb):
    n, d = x.shape
    e_total = edge_index.shape[1]
    bsz = length.shape[0]
    lsz = n // bsz
    h_heads, c_ch = a_src1.shape[1], a_src1.shape[2]
    f1 = W1.shape[1]

    src = edge_index[0]
    dst = edge_index[1]

    eye = jnp.eye(h_heads, dtype=F32)
    S = (a_src1[0][:, :, None] * eye[:, None, :]).reshape(f1, h_heads)
    Dm = (a_dst1[0][:, :, None] * eye[:, None, :]).reshape(f1, h_heads)
    R = jnp.repeat(eye, c_ch, axis=1)

    P1 = _tc_prep1(x, W1, S, Dm)

    z1 = jnp.zeros((n // 2 // 16, 128), F32)
    Th, Te = _sc_pass1(P1, src, dst, z1, n, e_total)

    T0 = Th[0].reshape(n, 64)
    T1 = Th[1].reshape(n, 64)
    D0 = Te[0][:, 0:64].reshape(n, 4)
    D1 = Te[1][:, 0:64].reshape(n, 4)

    h2c = _tc_mid(T0, T1, D0, D1, P1, R, b1.reshape(1, f1), W2)

    cvec = jnp.concatenate(
        [a_src2.reshape(1), a_dst2.reshape(1), jnp.zeros((14,), F32)])
    z2 = jnp.zeros((128, 128), F32)
    U = _sc_pass2(h2c.reshape(n // 16, 16), cvec, src, dst, z2, n, e_total)
    U0 = U[0].reshape(n, 16)
    U1 = U[1].reshape(n, 16)

    asc2 = a_src2.reshape(1, 1)
    adc2 = a_dst2.reshape(1, 1)
    out1, x_ = _tc_final(x, W_emb.T, b_emb.reshape(1, d), h2c, asc2, adc2,
                         U0, U1, mask.reshape(n, 1), b2.reshape(1, 1),
                         bsz, lsz)
    return (out1, x_)


# speedup vs baseline: 1.2810x; 1.2781x over previous
import functools

import jax
import jax.numpy as jnp
from jax import lax
from jax.experimental import pallas as pl
from jax.experimental.pallas import tpu as pltpu
from jax.experimental.pallas import tpu_sc as plsc

F32 = jnp.float32


def _tc_prep1(x, W1, S, Dm):
    n, d = x.shape
    f1 = W1.shape[1]
    blk = 256
    grid = n // blk

    def body(x_ref, w_ref, s_ref, dm_ref, out_ref):
        h1 = jnp.dot(x_ref[...], w_ref[...], preferred_element_type=F32)
        as1 = jnp.dot(h1, s_ref[...], preferred_element_type=F32)
        ad1 = jnp.dot(h1, dm_ref[...], preferred_element_type=F32)
        pad = jnp.zeros((blk, 56), F32)
        out_ref[...] = jnp.concatenate([h1, as1, ad1, pad], axis=1)

    return pl.pallas_call(
        body,
        grid=(grid,),
        in_specs=[
            pl.BlockSpec((blk, d), lambda i: (i, 0)),
            pl.BlockSpec((d, f1), lambda i: (0, 0)),
            pl.BlockSpec((f1, 4), lambda i: (0, 0)),
            pl.BlockSpec((f1, 4), lambda i: (0, 0)),
        ],
        out_specs=pl.BlockSpec((blk, 128), lambda i: (i, 0)),
        out_shape=jax.ShapeDtypeStruct((n, 128), F32),
    )(x, W1, S, Dm)


def _sc_pass1(P1, src, dst, zrows, n, e_total):
    info = plsc.get_sparse_core_info()
    nc, ns = info.num_cores, info.num_subcores
    ept = e_total // (nc * ns)
    K = 32
    nb = ept // K
    hrows = n // 2
    erows = n // 16
    hrpt = hrows // ns
    erpt = erows // ns

    mesh = plsc.VectorSubcoreMesh(core_axis_name="c", subcore_axis_name="s")

    @functools.partial(
        pl.kernel,
        out_type=[
            jax.ShapeDtypeStruct((nc, hrows, 128), F32),
            jax.ShapeDtypeStruct((nc, erows, 128), F32),
        ],
        mesh=mesh,
        scratch_types=[
            pltpu.VMEM((8 * K + 16,), jnp.int32),
            pltpu.VMEM((8 * K + 16,), jnp.int32),
            pltpu.VMEM((2, K), jnp.int32),
            pltpu.VMEM((2, K), jnp.int32),
            pltpu.VMEM((2, K, 128), F32),
            pltpu.VMEM((2, K, 128), F32),
            pltpu.VMEM((2, K, 128), F32),
            pltpu.VMEM((2, K, 128), F32),
            pltpu.VMEM_SHARED((hrows, 128), F32),
            pltpu.VMEM_SHARED((erows, 128), F32),
            pltpu.SemaphoreType.DMA,
            pltpu.SemaphoreType.DMA,
            pltpu.SemaphoreType.DMA,
            pltpu.SemaphoreType.DMA,
        ],
    )
    def k(p1_h, src_h, dst_h, z_h, outh_h, oute_h,
          sidx_s, sidx_d, idx_h, idx_e, gbuf_s, gbuf_d, obuf_h, obuf_e,
          acc_h, acc_e, sg0, sg1, ss0, ss1):
        c = lax.axis_index("c")
        s = lax.axis_index("s")
        SBK = 8 * K
        pltpu.sync_copy(z_h, acc_h.at[pl.ds(s * hrpt, hrpt)])
        pltpu.sync_copy(z_h.at[pl.ds(0, erpt)], acc_e.at[pl.ds(s * erpt, erpt)])
        pltpu.sync_copy(z_h.at[pl.ds(0, K)], obuf_e.at[0])
        pltpu.sync_copy(z_h.at[pl.ds(0, K)], obuf_e.at[1])
        plsc.subcore_barrier()

        lanes = lax.iota(jnp.int32, 16)
        zv = jnp.zeros((16,), F32)
        base = (c * ns + s) * ept
        sg = (sg0, sg1)
        ss = (ss0, ss1)

        def issue_gather(slot, o):
            pltpu.async_copy(p1_h.at[sidx_s.at[pl.ds(o, K)]], gbuf_s.at[slot],
                             sg[slot])
            pltpu.async_copy(p1_h.at[sidx_d.at[pl.ds(o, K)]], gbuf_d.at[slot],
                             sg[slot])

        def wait_gathers(slot, o):
            pltpu.make_async_copy(p1_h.at[sidx_s.at[pl.ds(o, K)]],
                                  gbuf_s.at[slot], sg[slot]).wait()
            pltpu.make_async_copy(p1_h.at[sidx_d.at[pl.ds(o, K)]],
                                  gbuf_d.at[slot], sg[slot]).wait()

        def issue_scatters(slot):
            pltpu.async_copy(obuf_h.at[slot], acc_h.at[idx_h.at[slot]],
                             ss[slot], add=True)
            pltpu.async_copy(obuf_e.at[slot], acc_e.at[idx_e.at[slot]],
                             ss[slot], add=True)

        def wait_scatters(slot):
            pltpu.make_async_copy(obuf_h.at[slot], acc_h.at[idx_h.at[slot]],
                                  ss[slot]).wait()
            pltpu.make_async_copy(obuf_e.at[slot], acc_e.at[idx_e.at[slot]],
                                  ss[slot]).wait()

        def compute(slot, o):
            for j in range(K // 16):
                dv = sidx_d[pl.ds(o + j * 16, 16)]
                idx_h[slot, pl.ds(j * 16, 16)] = lax.shift_right_logical(dv, 1)
                idx_e[slot, pl.ds(j * 16, 16)] = lax.shift_right_logical(dv, 4)

            @plsc.parallel_loop(0, K, 1, unroll=4)
            def edge(i):
                d = sidx_d[pl.ds(o + i, 16)][0]
                av = (gbuf_s[slot, i, pl.ds(64, 16)]
                      + gbuf_d[slot, i, pl.ds(68, 16)])
                av = jnp.maximum(av, 0.2 * av)
                e = jnp.exp(av)
                e = jnp.where(lanes < 4, e, 0.0)
                half = (d & 1) * 64
                ohalf = 64 - half
                obuf_h[slot, i, pl.ds(half + 0, 16)] = (
                    gbuf_s[slot, i, pl.ds(0, 16)] * e[0])
                obuf_h[slot, i, pl.ds(half + 16, 16)] = (
                    gbuf_s[slot, i, pl.ds(16, 16)] * e[1])
                obuf_h[slot, i, pl.ds(half + 32, 16)] = (
                    gbuf_s[slot, i, pl.ds(32, 16)] * e[2])
                obuf_h[slot, i, pl.ds(half + 48, 16)] = (
                    gbuf_s[slot, i, pl.ds(48, 16)] * e[3])
                obuf_h[slot, i, pl.ds(ohalf + 0, 16)] = zv
                obuf_h[slot, i, pl.ds(ohalf + 16, 16)] = zv
                obuf_h[slot, i, pl.ds(ohalf + 32, 16)] = zv
                obuf_h[slot, i, pl.ds(ohalf + 48, 16)] = zv
                obuf_e[slot, i, pl.ds(0, 16)] = zv
                obuf_e[slot, i, pl.ds(16, 16)] = zv
                obuf_e[slot, i, pl.ds(32, 16)] = zv
                obuf_e[slot, i, pl.ds(48, 16)] = zv
                col = (d & 15) * 4
                obuf_e[slot, i, pl.ds(col, 16)] = e

        nbs = nb // 8

        def run_super(g, carry):
            sb = base + g * SBK
            pltpu.sync_copy(src_h.at[pl.ds(sb, SBK)],
                            sidx_s.at[pl.ds(0, SBK)])
            pltpu.sync_copy(dst_h.at[pl.ds(sb, SBK)],
                            sidx_d.at[pl.ds(0, SBK)])
            issue_gather(0, 0)
            for p in range(4):
                o0 = (2 * p) * K
                o1 = o0 + K
                issue_gather(1, o1)
                wait_gathers(0, o0)
                if p == 0:
                    pl.when(g > 0)(lambda: wait_scatters(0))
                else:
                    wait_scatters(0)
                compute(0, o0)
                issue_scatters(0)
                if p < 3:
                    issue_gather(0, o1 + K)
                wait_gathers(1, o1)
                if p == 0:
                    pl.when(g > 0)(lambda: wait_scatters(1))
                else:
                    wait_scatters(1)
                compute(1, o1)
                issue_scatters(1)
            return carry

        lax.fori_loop(0, nbs, run_super, 0)
        wait_scatters(0)
        wait_scatters(1)
        plsc.subcore_barrier()
        pltpu.sync_copy(acc_h.at[pl.ds(s * hrpt, hrpt)],
                        outh_h.at[c, pl.ds(s * hrpt, hrpt)])
        pltpu.sync_copy(acc_e.at[pl.ds(s * erpt, erpt)],
                        oute_h.at[c, pl.ds(s * erpt, erpt)])

    return k(P1, src, dst, zrows)


def _tc_mid(T0, T1, D0, D1, P1, R, b1, W2):
    n = P1.shape[0]
    f1 = W2.shape[0]
    blk = 256
    grid = n // blk

    def body(t0_ref, t1_ref, d0_ref, d1_ref, p1_ref, r_ref, b1_ref, w2_ref,
             h2_ref):
        h1 = p1_ref[:, 0:64]
        as1 = p1_ref[:, 64:68]
        ad1 = p1_ref[:, 68:72]
        al = as1 + ad1
        al = jnp.maximum(al, 0.2 * al)
        es = jnp.exp(al)
        es64 = jnp.dot(es, r_ref[...], preferred_element_type=F32)
        num = t0_ref[...] + t1_ref[...] + h1 * es64
        den = d0_ref[...] + d1_ref[...] + es
        den64 = jnp.dot(den, r_ref[...], preferred_element_type=F32)
        g1 = jnp.maximum(num / (den64 + 1e-16) + b1_ref[...], 0.0)
        h2_ref[...] = jnp.dot(g1, w2_ref[...], preferred_element_type=F32)

    return pl.pallas_call(
        body,
        grid=(grid,),
        in_specs=[
            pl.BlockSpec((blk, 64), lambda i: (i, 0)),
            pl.BlockSpec((blk, 64), lambda i: (i, 0)),
            pl.BlockSpec((blk, 4), lambda i: (i, 0)),
            pl.BlockSpec((blk, 4), lambda i: (i, 0)),
            pl.BlockSpec((blk, 128), lambda i: (i, 0)),
            pl.BlockSpec((4, 64), lambda i: (0, 0)),
            pl.BlockSpec((1, 64), lambda i: (0, 0)),
            pl.BlockSpec((f1, 1), lambda i: (0, 0)),
        ],
        out_specs=pl.BlockSpec((blk, 1), lambda i: (i, 0)),
        out_shape=jax.ShapeDtypeStruct((n, 1), F32),
    )(T0, T1, D0, D1, P1, R, b1, W2)


def _sc_pass2(h2f, cvec, src, dst, zrows, n, e_total):
    info = plsc.get_sparse_core_info()
    nc, ns = info.num_cores, info.num_subcores
    ept = e_total // (nc * ns)
    K = 128
    nb = ept // K
    arows = n // 8
    arpt = arows // ns

    mesh = plsc.VectorSubcoreMesh(core_axis_name="c", subcore_axis_name="s")

    @functools.partial(
        pl.kernel,
        out_type=jax.ShapeDtypeStruct((nc, arows, 128), F32),
        mesh=mesh,
        compiler_params=pltpu.CompilerParams(needs_layout_passes=False,
                                             use_tc_tiling_on_sc=False),
        scratch_types=[
            pltpu.VMEM((K,), jnp.int32),
            pltpu.VMEM((K,), jnp.int32),
            pltpu.VMEM((K,), jnp.int32),
            pltpu.VMEM((K, 128), F32),
            pltpu.VMEM((n // 16, 16), F32),
            pltpu.VMEM((16,), F32),
            pltpu.VMEM_SHARED((arows, 128), F32),
            pltpu.SemaphoreType.DMA,
        ],
    )
    def k(h2_h, cv_h, src_h, dst_h, z_h, out_h,
          idx_s, idx_d, idx_r, obuf, h2t, cbuf, acc, sem):
        c = lax.axis_index("c")
        s = lax.axis_index("s")
        pltpu.sync_copy(z_h.at[pl.ds(0, arpt)], acc.at[pl.ds(s * arpt, arpt)])
        pltpu.sync_copy(z_h, obuf)
        pltpu.sync_copy(h2_h, h2t)
        pltpu.sync_copy(cv_h, cbuf)
        plsc.subcore_barrier()

        lanes = lax.iota(jnp.int32, 16)
        zv = jnp.zeros((16,), F32)
        cb = cbuf[...]
        c1 = cb[0]
        c2 = cb[1]
        base = (c * ns + s) * ept

        def run_batch(b, carry):
            off = base + b * K
            pltpu.sync_copy(src_h.at[pl.ds(off, K)], idx_s)
            pltpu.sync_copy(dst_h.at[pl.ds(off, K)], idx_d)
            for j in range(K // 16):
                rows = lanes + (j * 16)
                sv = idx_s[pl.ds(j * 16, 16)]
                dv = idx_d[pl.ds(j * 16, 16)]
                idx_r[pl.ds(j * 16, 16)] = lax.shift_right_logical(dv, 3)
                h2s = plsc.load_gather(
                    h2t, [lax.shift_right_logical(sv, 4), sv & 15])
                h2d = plsc.load_gather(
                    h2t, [lax.shift_right_logical(dv, 4), dv & 15])
                av = c1 * h2s + c2 * h2d
                av = jnp.maximum(av, 0.2 * av)
                e = jnp.exp(av)
                colv = (dv & 7) * 16
                plsc.store_scatter(obuf, [rows, colv], e * h2s)
                plsc.store_scatter(obuf, [rows, colv + 1], e)
            pltpu.sync_copy(obuf, acc.at[idx_r], add=True)
            for j in range(K // 16):
                rows = lanes + (j * 16)
                dv = idx_d[pl.ds(j * 16, 16)]
                colv = (dv & 7) * 16
                plsc.store_scatter(obuf, [rows, colv], zv)
                plsc.store_scatter(obuf, [rows, colv + 1], zv)
            return carry

        lax.fori_loop(0, nb, run_batch, 0)
        plsc.subcore_barrier()
        pltpu.sync_copy(acc.at[pl.ds(s * arpt, arpt)],
                        out_h.at[c, pl.ds(s * arpt, arpt)])

    return k(h2f, cvec, src, dst, zrows)


def _tc_final(x, WT, b_emb, h2c, asc2, adc2, U0, U1, maskf, b2, bsz, lsz):
    n, d = x.shape

    def body(x_ref, wt_ref, be_ref, h2_ref, asc_ref, adc_ref, u0_ref, u1_ref,
             m_ref, b2_ref, out1_ref, xo_ref):
        h2 = h2_ref[...]
        al = h2 * (asc_ref[0, 0] + adc_ref[0, 0])
        al = jnp.maximum(al, 0.2 * al)
        es = jnp.exp(al)
        num = u0_ref[:, 0:1] + u1_ref[:, 0:1] + es * h2
        den = u0_ref[:, 1:2] + u1_ref[:, 1:2] + es
        z = num / (den + 1e-16) + b2_ref[0, 0]
        att = 1.0 / (1.0 + jnp.exp(-z))
        emb = jnp.dot(x_ref[...], wt_ref[...], preferred_element_type=F32)
        emb = jnp.maximum(emb + be_ref[...], 0.0)
        xv = att * emb
        xo_ref[...] = xv
        m = m_ref[...]
        pmax = jnp.max(xv + (m - 1.0) * 1e9, axis=0)
        pmean = jnp.sum(xv * m, axis=0) / jnp.sum(m)
        g = pl.program_id(0)
        out1_ref[pl.ds(g, 1), :] = (pmax + pmean)[None, :]

    return pl.pallas_call(
        body,
        grid=(bsz,),
        in_specs=[
            pl.BlockSpec((lsz, d), lambda i: (i, 0)),
            pl.BlockSpec((d, d), lambda i: (0, 0)),
            pl.BlockSpec((1, d), lambda i: (0, 0)),
            pl.BlockSpec((lsz, 1), lambda i: (i, 0)),
            pl.BlockSpec((1, 1), lambda i: (0, 0)),
            pl.BlockSpec((1, 1), lambda i: (0, 0)),
            pl.BlockSpec((lsz, 16), lambda i: (i, 0)),
            pl.BlockSpec((lsz, 16), lambda i: (i, 0)),
            pl.BlockSpec((lsz, 1), lambda i: (i, 0)),
            pl.BlockSpec((1, 1), lambda i: (0, 0)),
        ],
        out_specs=[
            pl.BlockSpec((bsz, d), lambda i: (0, 0)),
            pl.BlockSpec((lsz, d), lambda i: (i, 0)),
        ],
        out_shape=[
            jax.ShapeDtypeStruct((bsz, d), F32),
            jax.ShapeDtypeStruct((n, d), F32),
        ],
    )(x, WT, b_emb, h2c, asc2, adc2, U0, U1, maskf, b2)


def kernel(x, mask, edge_index, length, W1, a_src1, a_dst1, b1,
           W2, a_src2, a_dst2, b2, W_emb, b_emb):
    n, d = x.shape
    e_total = edge_index.shape[1]
    bsz = length.shape[0]
    lsz = n // bsz
    h_heads, c_ch = a_src1.shape[1], a_src1.shape[2]
    f1 = W1.shape[1]

    src = edge_index[0]
    dst = edge_index[1]

    eye = jnp.eye(h_heads, dtype=F32)
    S = (a_src1[0][:, :, None] * eye[:, None, :]).reshape(f1, h_heads)
    Dm = (a_dst1[0][:, :, None] * eye[:, None, :]).reshape(f1, h_heads)
    R = jnp.repeat(eye, c_ch, axis=1)

    P1 = _tc_prep1(x, W1, S, Dm)

    z1 = jnp.zeros((n // 2 // 16, 128), F32)
    Th, Te = _sc_pass1(P1, src, dst, z1, n, e_total)

    T0 = Th[0].reshape(n, 64)
    T1 = Th[1].reshape(n, 64)
    D0 = Te[0][:, 0:64].reshape(n, 4)
    D1 = Te[1][:, 0:64].reshape(n, 4)

    h2c = _tc_mid(T0, T1, D0, D1, P1, R, b1.reshape(1, f1), W2)

    cvec = jnp.concatenate(
        [a_src2.reshape(1), a_dst2.reshape(1), jnp.zeros((14,), F32)])
    z2 = jnp.zeros((128, 128), F32)
    U = _sc_pass2(h2c.reshape(n // 16, 16), cvec, src, dst, z2, n, e_total)
    U0 = U[0].reshape(n, 16)
    U1 = U[1].reshape(n, 16)

    asc2 = a_src2.reshape(1, 1)
    adc2 = a_dst2.reshape(1, 1)
    out1, x_ = _tc_final(x, W_emb.T, b_emb.reshape(1, d), h2c, asc2, adc2,
                         U0, U1, mask.reshape(n, 1), b2.reshape(1, 1),
                         bsz, lsz)
    return (out1, x_)


# speedup vs baseline: 1.4527x; 1.1340x over previous
import functools

import jax
import jax.numpy as jnp
from jax import lax
from jax.experimental import pallas as pl
from jax.experimental.pallas import tpu as pltpu
from jax.experimental.pallas import tpu_sc as plsc

F32 = jnp.float32


def _tc_prep1(x, W1, S, Dm):
    n, d = x.shape
    f1 = W1.shape[1]
    blk = 256
    grid = n // blk

    def body(x_ref, w_ref, s_ref, dm_ref, out_ref):
        h1 = jnp.dot(x_ref[...], w_ref[...], preferred_element_type=F32)
        as1 = jnp.dot(h1, s_ref[...], preferred_element_type=F32)
        ad1 = jnp.dot(h1, dm_ref[...], preferred_element_type=F32)
        pad = jnp.zeros((blk, 56), F32)
        out_ref[...] = jnp.concatenate([h1, as1, ad1, pad], axis=1)

    return pl.pallas_call(
        body,
        grid=(grid,),
        in_specs=[
            pl.BlockSpec((blk, d), lambda i: (i, 0)),
            pl.BlockSpec((d, f1), lambda i: (0, 0)),
            pl.BlockSpec((f1, 4), lambda i: (0, 0)),
            pl.BlockSpec((f1, 4), lambda i: (0, 0)),
        ],
        out_specs=pl.BlockSpec((blk, 128), lambda i: (i, 0)),
        out_shape=jax.ShapeDtypeStruct((n, 128), F32),
    )(x, W1, S, Dm)


def _sc_pass1(P1, src, dst, zrows, n, e_total):
    info = plsc.get_sparse_core_info()
    nc, ns = info.num_cores, info.num_subcores
    ept = e_total // (nc * ns)
    K = 32
    nb = ept // K
    hrows = n // 2
    erows = n // 16
    hrpt = hrows // ns
    erpt = erows // ns

    mesh = plsc.VectorSubcoreMesh(core_axis_name="c", subcore_axis_name="s")

    @functools.partial(
        pl.kernel,
        out_type=[
            jax.ShapeDtypeStruct((nc, hrows, 128), F32),
            jax.ShapeDtypeStruct((nc, erows, 128), F32),
        ],
        mesh=mesh,
        scratch_types=[
            pltpu.VMEM((8 * K + 16,), jnp.int32),
            pltpu.VMEM((8 * K + 16,), jnp.int32),
            pltpu.VMEM((2, K), jnp.int32),
            pltpu.VMEM((2, K), jnp.int32),
            pltpu.VMEM((2, K, 128), F32),
            pltpu.VMEM((2, K, 128), F32),
            pltpu.VMEM((2, K, 128), F32),
            pltpu.VMEM((2, K, 128), F32),
            pltpu.VMEM_SHARED((hrows, 128), F32),
            pltpu.VMEM_SHARED((erows, 128), F32),
            pltpu.SemaphoreType.DMA,
            pltpu.SemaphoreType.DMA,
            pltpu.SemaphoreType.DMA,
            pltpu.SemaphoreType.DMA,
        ],
    )
    def k(p1_h, src_h, dst_h, z_h, outh_h, oute_h,
          sidx_s, sidx_d, idx_h, idx_e, gbuf_s, gbuf_d, obuf_h, obuf_e,
          acc_h, acc_e, sg0, sg1, ss0, ss1):
        c = lax.axis_index("c")
        s = lax.axis_index("s")
        SBK = 8 * K
        pltpu.sync_copy(z_h, acc_h.at[pl.ds(s * hrpt, hrpt)])
        pltpu.sync_copy(z_h.at[pl.ds(0, erpt)], acc_e.at[pl.ds(s * erpt, erpt)])
        pltpu.sync_copy(z_h.at[pl.ds(0, K)], obuf_e.at[0])
        pltpu.sync_copy(z_h.at[pl.ds(0, K)], obuf_e.at[1])
        plsc.subcore_barrier()

        lanes = lax.iota(jnp.int32, 16)
        zv = jnp.zeros((16,), F32)
        base = (c * ns + s) * ept
        sg = (sg0, sg1)
        ss = (ss0, ss1)

        def issue_gather(slot, o):
            pltpu.async_copy(p1_h.at[sidx_s.at[pl.ds(o, K)]], gbuf_s.at[slot],
                             sg[slot])
            pltpu.async_copy(p1_h.at[sidx_d.at[pl.ds(o, K)]], gbuf_d.at[slot],
                             sg[slot])

        def wait_gathers(slot, o):
            pltpu.make_async_copy(p1_h.at[sidx_s.at[pl.ds(o, K)]],
                                  gbuf_s.at[slot], sg[slot]).wait()
            pltpu.make_async_copy(p1_h.at[sidx_d.at[pl.ds(o, K)]],
                                  gbuf_d.at[slot], sg[slot]).wait()

        def issue_scatters(slot):
            pltpu.async_copy(obuf_h.at[slot], acc_h.at[idx_h.at[slot]],
                             ss[slot], add=True)
            pltpu.async_copy(obuf_e.at[slot], acc_e.at[idx_e.at[slot]],
                             ss[slot], add=True)

        def wait_scatters(slot):
            pltpu.make_async_copy(obuf_h.at[slot], acc_h.at[idx_h.at[slot]],
                                  ss[slot]).wait()
            pltpu.make_async_copy(obuf_e.at[slot], acc_e.at[idx_e.at[slot]],
                                  ss[slot]).wait()

        def compute(slot, o):
            for j in range(K // 16):
                dv = sidx_d[pl.ds(o + j * 16, 16)]
                idx_h[slot, pl.ds(j * 16, 16)] = lax.shift_right_logical(dv, 1)
                idx_e[slot, pl.ds(j * 16, 16)] = lax.shift_right_logical(dv, 4)

            @plsc.parallel_loop(0, K, 1, unroll=4)
            def edge(i):
                d = sidx_d[pl.ds(o + i, 16)][0]
                av = (gbuf_s[slot, i, pl.ds(64, 16)]
                      + gbuf_d[slot, i, pl.ds(68, 16)])
                av = jnp.maximum(av, 0.2 * av)
                e = jnp.exp(av)
                e = jnp.where(lanes < 4, e, 0.0)
                half = (d & 1) * 64
                ohalf = 64 - half
                obuf_h[slot, i, pl.ds(half + 0, 16)] = (
                    gbuf_s[slot, i, pl.ds(0, 16)] * e[0])
                obuf_h[slot, i, pl.ds(half + 16, 16)] = (
                    gbuf_s[slot, i, pl.ds(16, 16)] * e[1])
                obuf_h[slot, i, pl.ds(half + 32, 16)] = (
                    gbuf_s[slot, i, pl.ds(32, 16)] * e[2])
                obuf_h[slot, i, pl.ds(half + 48, 16)] = (
                    gbuf_s[slot, i, pl.ds(48, 16)] * e[3])
                obuf_h[slot, i, pl.ds(ohalf + 0, 16)] = zv
                obuf_h[slot, i, pl.ds(ohalf + 16, 16)] = zv
                obuf_h[slot, i, pl.ds(ohalf + 32, 16)] = zv
                obuf_h[slot, i, pl.ds(ohalf + 48, 16)] = zv
                obuf_e[slot, i, pl.ds(0, 16)] = zv
                obuf_e[slot, i, pl.ds(16, 16)] = zv
                obuf_e[slot, i, pl.ds(32, 16)] = zv
                obuf_e[slot, i, pl.ds(48, 16)] = zv
                col = (d & 15) * 4
                obuf_e[slot, i, pl.ds(col, 16)] = e

        nbs = nb // 8

        def run_super(g, carry):
            sb = base + g * SBK
            pltpu.sync_copy(src_h.at[pl.ds(sb, SBK)],
                            sidx_s.at[pl.ds(0, SBK)])
            pltpu.sync_copy(dst_h.at[pl.ds(sb, SBK)],
                            sidx_d.at[pl.ds(0, SBK)])
            issue_gather(0, 0)
            for p in range(4):
                o0 = (2 * p) * K
                o1 = o0 + K
                issue_gather(1, o1)
                wait_gathers(0, o0)
                if p == 0:
                    pl.when(g > 0)(lambda: wait_scatters(0))
                else:
                    wait_scatters(0)
                compute(0, o0)
                issue_scatters(0)
                if p < 3:
                    issue_gather(0, o1 + K)
                wait_gathers(1, o1)
                if p == 0:
                    pl.when(g > 0)(lambda: wait_scatters(1))
                else:
                    wait_scatters(1)
                compute(1, o1)
                issue_scatters(1)
            return carry

        lax.fori_loop(0, nbs, run_super, 0)
        wait_scatters(0)
        wait_scatters(1)
        plsc.subcore_barrier()
        pltpu.sync_copy(acc_h.at[pl.ds(s * hrpt, hrpt)],
                        outh_h.at[c, pl.ds(s * hrpt, hrpt)])
        pltpu.sync_copy(acc_e.at[pl.ds(s * erpt, erpt)],
                        oute_h.at[c, pl.ds(s * erpt, erpt)])

    return k(P1, src, dst, zrows)


def _tc_mid(T0, T1, D0, D1, P1, R, b1, W2):
    n = P1.shape[0]
    f1 = W2.shape[0]
    blk = 256
    grid = n // blk

    def body(t0_ref, t1_ref, d0_ref, d1_ref, p1_ref, r_ref, b1_ref, w2_ref,
             h2_ref):
        h1 = p1_ref[:, 0:64]
        as1 = p1_ref[:, 64:68]
        ad1 = p1_ref[:, 68:72]
        al = as1 + ad1
        al = jnp.maximum(al, 0.2 * al)
        es = jnp.exp(al)
        es64 = jnp.dot(es, r_ref[...], preferred_element_type=F32)
        num = t0_ref[...] + t1_ref[...] + h1 * es64
        den = d0_ref[...] + d1_ref[...] + es
        den64 = jnp.dot(den, r_ref[...], preferred_element_type=F32)
        g1 = jnp.maximum(num / (den64 + 1e-16) + b1_ref[...], 0.0)
        h2_ref[...] = jnp.dot(g1, w2_ref[...], preferred_element_type=F32)

    return pl.pallas_call(
        body,
        grid=(grid,),
        in_specs=[
            pl.BlockSpec((blk, 64), lambda i: (i, 0)),
            pl.BlockSpec((blk, 64), lambda i: (i, 0)),
            pl.BlockSpec((blk, 4), lambda i: (i, 0)),
            pl.BlockSpec((blk, 4), lambda i: (i, 0)),
            pl.BlockSpec((blk, 128), lambda i: (i, 0)),
            pl.BlockSpec((4, 64), lambda i: (0, 0)),
            pl.BlockSpec((1, 64), lambda i: (0, 0)),
            pl.BlockSpec((f1, 1), lambda i: (0, 0)),
        ],
        out_specs=pl.BlockSpec((blk, 1), lambda i: (i, 0)),
        out_shape=jax.ShapeDtypeStruct((n, 1), F32),
    )(T0, T1, D0, D1, P1, R, b1, W2)


def _sc_pass2(h2f, cvec, src, dst, zrows, n, e_total):
    info = plsc.get_sparse_core_info()
    nc, ns = info.num_cores, info.num_subcores
    ept = e_total // (nc * ns)
    K = 128
    nb = ept // K
    SBK = 4 * K
    arows = n // 8
    arpt = arows // ns

    mesh = plsc.VectorSubcoreMesh(core_axis_name="c", subcore_axis_name="s")

    @functools.partial(
        pl.kernel,
        out_type=jax.ShapeDtypeStruct((nc, arows, 128), F32),
        mesh=mesh,
        compiler_params=pltpu.CompilerParams(needs_layout_passes=False,
                                             use_tc_tiling_on_sc=False),
        scratch_types=[
            pltpu.VMEM((SBK + 16,), jnp.int32),
            pltpu.VMEM((SBK + 16,), jnp.int32),
            pltpu.VMEM((2, K), jnp.int32),
            pltpu.VMEM((2, K), jnp.int32),
            pltpu.VMEM((2, K, 128), F32),
            pltpu.VMEM((n // 16, 16), F32),
            pltpu.VMEM((16,), F32),
            pltpu.VMEM_SHARED((arows, 128), F32),
            pltpu.SemaphoreType.DMA,
            pltpu.SemaphoreType.DMA,
        ],
    )
    def k(h2_h, cv_h, src_h, dst_h, z_h, out_h,
          sidx_s, sidx_d, idx_r, colb, obuf, h2t, cbuf, acc, ss0, ss1):
        c = lax.axis_index("c")
        s = lax.axis_index("s")
        pltpu.sync_copy(z_h.at[pl.ds(0, arpt)], acc.at[pl.ds(s * arpt, arpt)])
        pltpu.sync_copy(z_h, obuf.at[0])
        pltpu.sync_copy(z_h, obuf.at[1])
        pltpu.sync_copy(h2_h, h2t)
        pltpu.sync_copy(cv_h, cbuf)
        plsc.subcore_barrier()

        lanes = lax.iota(jnp.int32, 16)
        zv = jnp.zeros((16,), F32)
        cb = cbuf[...]
        c1 = cb[0]
        c2 = cb[1]
        base = (c * ns + s) * ept
        ss = (ss0, ss1)

        def issue_scatter(slot):
            pltpu.async_copy(obuf.at[slot], acc.at[idx_r.at[slot]],
                             ss[slot], add=True)

        def wait_scatter(slot):
            pltpu.make_async_copy(obuf.at[slot], acc.at[idx_r.at[slot]],
                                  ss[slot]).wait()

        def drain_restore(slot):
            wait_scatter(slot)
            sl = lanes * 0 + slot
            for j in range(K // 16):
                rows = lanes + (j * 16)
                colv = colb[slot, pl.ds(j * 16, 16)]
                plsc.store_scatter(obuf, [sl, rows, colv], zv)
                plsc.store_scatter(obuf, [sl, rows, colv + 1], zv)

        def compute(slot, o):
            sl = lanes * 0 + slot
            for j in range(K // 16):
                rows = lanes + (j * 16)
                sv = sidx_s[pl.ds(o + j * 16, 16)]
                dv = sidx_d[pl.ds(o + j * 16, 16)]
                idx_r[slot, pl.ds(j * 16, 16)] = lax.shift_right_logical(dv, 3)
                colv = (dv & 7) * 16
                colb[slot, pl.ds(j * 16, 16)] = colv
                h2s = plsc.load_gather(
                    h2t, [lax.shift_right_logical(sv, 4), sv & 15])
                h2d = plsc.load_gather(
                    h2t, [lax.shift_right_logical(dv, 4), dv & 15])
                av = c1 * h2s + c2 * h2d
                av = jnp.maximum(av, 0.2 * av)
                e = jnp.exp(av)
                plsc.store_scatter(obuf, [sl, rows, colv], e * h2s)
                plsc.store_scatter(obuf, [sl, rows, colv + 1], e)
            issue_scatter(slot)

        nbs = nb // 4

        def run_super(g, carry):
            sb = base + g * SBK
            pltpu.sync_copy(src_h.at[pl.ds(sb, SBK)],
                            sidx_s.at[pl.ds(0, SBK)])
            pltpu.sync_copy(dst_h.at[pl.ds(sb, SBK)],
                            sidx_d.at[pl.ds(0, SBK)])
            for p in range(2):
                o0 = (2 * p) * K
                if p == 0:
                    pl.when(g > 0)(lambda: drain_restore(0))
                else:
                    drain_restore(0)
                compute(0, o0)
                if p == 0:
                    pl.when(g > 0)(lambda: drain_restore(1))
                else:
                    drain_restore(1)
                compute(1, o0 + K)
            return carry

        lax.fori_loop(0, nbs, run_super, 0)
        wait_scatter(0)
        wait_scatter(1)
        plsc.subcore_barrier()
        pltpu.sync_copy(acc.at[pl.ds(s * arpt, arpt)],
                        out_h.at[c, pl.ds(s * arpt, arpt)])

    return k(h2f, cvec, src, dst, zrows)


def _tc_final(x, WT, b_emb, h2c, asc2, adc2, U0, U1, maskf, b2, bsz, lsz):
    n, d = x.shape

    def body(x_ref, wt_ref, be_ref, h2_ref, asc_ref, adc_ref, u0_ref, u1_ref,
             m_ref, b2_ref, out1_ref, xo_ref):
        h2 = h2_ref[...]
        al = h2 * (asc_ref[0, 0] + adc_ref[0, 0])
        al = jnp.maximum(al, 0.2 * al)
        es = jnp.exp(al)
        num = u0_ref[:, 0:1] + u1_ref[:, 0:1] + es * h2
        den = u0_ref[:, 1:2] + u1_ref[:, 1:2] + es
        z = num / (den + 1e-16) + b2_ref[0, 0]
        att = 1.0 / (1.0 + jnp.exp(-z))
        emb = jnp.dot(x_ref[...], wt_ref[...], preferred_element_type=F32)
        emb = jnp.maximum(emb + be_ref[...], 0.0)
        xv = att * emb
        xo_ref[...] = xv
        m = m_ref[...]
        pmax = jnp.max(xv + (m - 1.0) * 1e9, axis=0)
        pmean = jnp.sum(xv * m, axis=0) / jnp.sum(m)
        g = pl.program_id(0)
        out1_ref[pl.ds(g, 1), :] = (pmax + pmean)[None, :]

    return pl.pallas_call(
        body,
        grid=(bsz,),
        in_specs=[
            pl.BlockSpec((lsz, d), lambda i: (i, 0)),
            pl.BlockSpec((d, d), lambda i: (0, 0)),
            pl.BlockSpec((1, d), lambda i: (0, 0)),
            pl.BlockSpec((lsz, 1), lambda i: (i, 0)),
            pl.BlockSpec((1, 1), lambda i: (0, 0)),
            pl.BlockSpec((1, 1), lambda i: (0, 0)),
            pl.BlockSpec((lsz, 16), lambda i: (i, 0)),
            pl.BlockSpec((lsz, 16), lambda i: (i, 0)),
            pl.BlockSpec((lsz, 1), lambda i: (i, 0)),
            pl.BlockSpec((1, 1), lambda i: (0, 0)),
        ],
        out_specs=[
            pl.BlockSpec((bsz, d), lambda i: (0, 0)),
            pl.BlockSpec((lsz, d), lambda i: (i, 0)),
        ],
        out_shape=[
            jax.ShapeDtypeStruct((bsz, d), F32),
            jax.ShapeDtypeStruct((n, d), F32),
        ],
    )(x, WT, b_emb, h2c, asc2, adc2, U0, U1, maskf, b2)


def kernel(x, mask, edge_index, length, W1, a_src1, a_dst1, b1,
           W2, a_src2, a_dst2, b2, W_emb, b_emb):
    n, d = x.shape
    e_total = edge_index.shape[1]
    bsz = length.shape[0]
    lsz = n // bsz
    h_heads, c_ch = a_src1.shape[1], a_src1.shape[2]
    f1 = W1.shape[1]

    src = edge_index[0]
    dst = edge_index[1]

    eye = jnp.eye(h_heads, dtype=F32)
    S = (a_src1[0][:, :, None] * eye[:, None, :]).reshape(f1, h_heads)
    Dm = (a_dst1[0][:, :, None] * eye[:, None, :]).reshape(f1, h_heads)
    R = jnp.repeat(eye, c_ch, axis=1)

    P1 = _tc_prep1(x, W1, S, Dm)

    z1 = jnp.zeros((n // 2 // 16, 128), F32)
    Th, Te = _sc_pass1(P1, src, dst, z1, n, e_total)

    T0 = Th[0].reshape(n, 64)
    T1 = Th[1].reshape(n, 64)
    D0 = Te[0][:, 0:64].reshape(n, 4)
    D1 = Te[1][:, 0:64].reshape(n, 4)

    h2c = _tc_mid(T0, T1, D0, D1, P1, R, b1.reshape(1, f1), W2)

    cvec = jnp.concatenate(
        [a_src2.reshape(1), a_dst2.reshape(1), jnp.zeros((14,), F32)])
    z2 = jnp.zeros((128, 128), F32)
    U = _sc_pass2(h2c.reshape(n // 16, 16), cvec, src, dst, z2, n, e_total)
    U0 = U[0].reshape(n, 16)
    U1 = U[1].reshape(n, 16)

    asc2 = a_src2.reshape(1, 1)
    adc2 = a_dst2.reshape(1, 1)
    out1, x_ = _tc_final(x, W_emb.T, b_emb.reshape(1, d), h2c, asc2, adc2,
                         U0, U1, mask.reshape(n, 1), b2.reshape(1, 1),
                         bsz, lsz)
    return (out1, x_)
